# baseline (speedup 1.0000x reference)
import functools

import jax
import jax.numpy as jnp
from jax import lax
from jax.experimental import pallas as pl
from jax.experimental.pallas import tpu as pltpu
from jax.experimental.pallas import tpu_sc as plsc

LAMBDA_FAKE_W = 1.0
B = 256
D = 150528
NW = 32
RPT = B // NW
CPR = 6
CH = D // CPR
NCH = RPT * CPR
UNROLL = 8
NIT = CH // (16 * UNROLL)


def _chunk_sum(rb, xb):
    def body(i, acc):
        base = i * (16 * UNROLL)
        v = []
        for k in range(UNROLL):
            rv = rb[pl.ds(base + k * 16, 16)]
            xv = xb[pl.ds(base + k * 16, 16)]
            v.append(jnp.abs(rv - xv))
        t = ((v[0] + v[1]) + (v[2] + v[3])) + ((v[4] + v[5]) + (v[6] + v[7]))
        return acc + t
    return lax.fori_loop(0, NIT, body, jnp.zeros((16,), jnp.float32))


def _sc_partials(r_hbm, x_hbm, y_hbm, out_hbm, rbuf, xbuf, ybuf, obuf,
                 rsem, xsem, ysem, osem):
    cid = lax.axis_index("c")
    sid = lax.axis_index("s")
    wid = sid * 2 + cid
    g0 = wid * NCH
    row0 = wid * RPT

    pltpu.async_copy(y_hbm.at[pl.ds(row0, RPT), :], ybuf, ysem).wait()

    handles = {}

    def start(c, slot):
        handles[slot] = (
            pltpu.async_copy(r_hbm.at[g0 + c], rbuf.at[slot], rsem.at[slot]),
            pltpu.async_copy(x_hbm.at[g0 + c], xbuf.at[slot], xsem.at[slot]),
        )

    def wait(slot):
        hr, hx = handles[slot]
        hr.wait()
        hx.wait()

    acc_real = jnp.zeros((16,), jnp.float32)
    acc_all = jnp.zeros((16,), jnp.float32)
    acc_cnt = jnp.zeros((16,), jnp.float32)

    start(0, 0)
    for c in range(NCH):
        if c + 1 < NCH:
            start(c + 1, (c + 1) % 2)
        wait(c % 2)
        cs = _chunk_sum(rbuf.at[c % 2], xbuf.at[c % 2])
        yrow = ybuf[c // CPR, :]
        acc_all = acc_all + cs
        acc_real = acc_real + cs * yrow
        if c % CPR == 0:
            acc_cnt = acc_cnt + yrow

    obuf[0, :] = acc_real
    obuf[1, :] = acc_all
    obuf[2, :] = acc_cnt
    pltpu.async_copy(obuf, out_hbm.at[wid], osem).wait()


_sc_call = functools.partial(
    pl.kernel,
    out_type=jax.ShapeDtypeStruct((NW, 3, 16), jnp.float32),
    mesh=plsc.VectorSubcoreMesh(core_axis_name="c", subcore_axis_name="s"),
    scratch_types=[
        pltpu.VMEM((2, CH), jnp.float32),
        pltpu.VMEM((2, CH), jnp.float32),
        pltpu.VMEM((RPT, 16), jnp.float32),
        pltpu.VMEM((3, 16), jnp.float32),
        pltpu.SemaphoreType.DMA((2,)),
        pltpu.SemaphoreType.DMA((2,)),
        pltpu.SemaphoreType.DMA,
        pltpu.SemaphoreType.DMA,
    ],
)(_sc_partials)


def kernel(recons, x, y):
    rc = recons.reshape(B * CPR, CH)
    xc = x.reshape(B * CPR, CH)
    y16 = jnp.broadcast_to(y.astype(jnp.float32)[:, None], (B, 16))

    parts = _sc_call(rc, xc, y16)
    sum_real = jnp.sum(parts[:, 0, :])
    sum_all = jnp.sum(parts[:, 1, :])
    n_real = jnp.sum(parts[:, 2, :]) / 16.0
    n_fake = B - n_real
    sum_fake = sum_all - sum_real
    loss_real = jnp.where(n_real > 0, sum_real / (n_real * D), 0.0)
    loss_fake = jnp.where(n_fake > 0, sum_fake / (n_fake * D), 0.0)
    return loss_real - LAMBDA_FAKE_W * loss_fake


# speedup vs baseline: 1.8803x; 1.8803x over previous
import jax
import jax.numpy as jnp
from jax.experimental import pallas as pl
from jax.experimental.pallas import tpu as pltpu

LAMBDA_FAKE_W = 1.0
B = 256
D = 150528
RB = 8
NC = 3
CW = 1176 // NC
NSTEPS = B // RB


def _loss_kernel(y_ref, r_ref, x_ref, o_ref, acc_ref):
    step = pl.program_id(0)
    cstep = pl.program_id(1)

    @pl.when(jnp.logical_and(step == 0, cstep == 0))
    def _init():
        acc_ref[0] = 0.0
        acc_ref[1] = 0.0
        acc_ref[2] = 0.0

    d = jnp.abs(r_ref[...] - x_ref[...])
    s = jnp.sum(d, axis=(1, 2)).reshape(RB, 1)
    yv = y_ref[...]
    acc_ref[0] += jnp.sum(s * yv)
    acc_ref[1] += jnp.sum(s)

    @pl.when(cstep == 0)
    def _count():
        acc_ref[2] += jnp.sum(yv)

    @pl.when(jnp.logical_and(step == NSTEPS - 1, cstep == NC - 1))
    def _finalize():
        n_real = acc_ref[2]
        n_fake = B - n_real
        sum_real = acc_ref[0]
        sum_fake = acc_ref[1] - sum_real
        loss_real = jnp.where(n_real > 0, sum_real / (n_real * D), 0.0)
        loss_fake = jnp.where(n_fake > 0, sum_fake / (n_fake * D), 0.0)
        o_ref[...] = (loss_real - LAMBDA_FAKE_W * loss_fake).reshape(1, 1)


def kernel(recons, x, y):
    r3 = recons.reshape(B, 1176, 128)
    x3 = x.reshape(B, 1176, 128)
    y2 = y.astype(jnp.float32).reshape(B, 1)

    out = pl.pallas_call(
        _loss_kernel,
        grid=(NSTEPS, NC),
        in_specs=[
            pl.BlockSpec((RB, 1), lambda i, j: (i, 0)),
            pl.BlockSpec((RB, CW, 128), lambda i, j: (i, j, 0)),
            pl.BlockSpec((RB, CW, 128), lambda i, j: (i, j, 0)),
        ],
        out_specs=pl.BlockSpec((1, 1), lambda i, j: (0, 0)),
        out_shape=jax.ShapeDtypeStruct((1, 1), jnp.float32),
        scratch_shapes=[pltpu.SMEM((3,), jnp.float32)],
        compiler_params=pltpu.CompilerParams(
            dimension_semantics=("arbitrary", "arbitrary"),
        ),
    )(y2, r3, x3)
    return out.reshape(())
